# baseline (speedup 1.0000x reference)
import functools

import jax
import jax.numpy as jnp
from jax import lax
from jax.experimental import pallas as pl
from jax.experimental.pallas import tpu as pltpu
from jax.experimental.pallas import tpu_sc as plsc

_L = 16
_CHUNK = 128


@functools.lru_cache(maxsize=None)
def _sc_gather_prod(total, rank, table_rows):
    info = plsc.get_sparse_core_info()
    num_workers = info.num_cores * info.num_subcores
    per_w = total // num_workers
    assert per_w % _CHUNK == 0
    n_chunks = per_w // _CHUNK

    mesh = plsc.VectorSubcoreMesh(core_axis_name="c", subcore_axis_name="s")

    @functools.partial(
        pl.kernel,
        mesh=mesh,
        compiler_params=pltpu.CompilerParams(use_tc_tiling_on_sc=False),
        out_type=jax.ShapeDtypeStruct((total, rank), jnp.bfloat16),
        scratch_types=[
            pltpu.VMEM((per_w,), jnp.float32),
            pltpu.VMEM((per_w,), jnp.float32),
            pltpu.VMEM((per_w,), jnp.float32),
            pltpu.VMEM((_CHUNK,), jnp.int32),
            pltpu.VMEM((_CHUNK,), jnp.int32),
            pltpu.VMEM((_CHUNK,), jnp.int32),
            pltpu.VMEM((_CHUNK, rank), jnp.bfloat16),
            pltpu.VMEM((_CHUNK, rank), jnp.bfloat16),
            pltpu.VMEM((_CHUNK, rank), jnp.bfloat16),
            pltpu.VMEM((_CHUNK, rank), jnp.bfloat16),
            pltpu.SemaphoreType.DMA,
        ],
    )
    def sc_fn(xf, yf, tf, At, Bt, Ct, p_out,
              fx, fy, ft, ix, iy, it, rA, rB, rC, pbuf, sem):
        wid = lax.axis_index("s") * info.num_cores + lax.axis_index("c")
        base = wid * per_w
        pltpu.sync_copy(xf.at[pl.ds(base, per_w)], fx)
        pltpu.sync_copy(yf.at[pl.ds(base, per_w)], fy)
        pltpu.sync_copy(tf.at[pl.ds(base, per_w)], ft)

        xscale = jnp.float32(table_rows - 1)
        yscale = jnp.float32(table_rows)
        hi = table_rows - 1
        row = pl.ds(0, rank)

        def chunk_body(c, carry):
            coff = c * _CHUNK
            for gi in range(_CHUNK // _L):
                src = pl.ds(coff + gi * _L, _L)
                dst = pl.ds(gi * _L, _L)
                ix[dst] = jnp.clip((fx[src] * xscale).astype(jnp.int32), 0, hi)
                iy[dst] = jnp.clip((fy[src] * yscale - 1.0).astype(jnp.int32), 0, hi)
                it[dst] = jnp.clip((ft[src] * yscale - 1.0).astype(jnp.int32), 0, hi)
            ca = pltpu.async_copy(At.at[ix], rA, sem)
            cb = pltpu.async_copy(Bt.at[iy], rB, sem)
            cc = pltpu.async_copy(Ct.at[it], rC, sem)
            ca.wait()
            cb.wait()
            cc.wait()
            for j in range(_CHUNK):
                pbuf[j, row] = rA[j, row] * rB[j, row] * rC[j, row]
            pltpu.sync_copy(pbuf, p_out.at[pl.ds(base + coff, _CHUNK)])
            return carry

        lax.fori_loop(0, n_chunks, chunk_body, 0)

    return sc_fn


@functools.lru_cache(maxsize=None)
def _tc_linear(n, k, feat):
    blk = 1024

    def mm(p_ref, w_ref, b_ref, o_ref):
        o_ref[...] = (
            jnp.dot(p_ref[...], w_ref[...], preferred_element_type=jnp.float32)
            + b_ref[...]
        )

    return pl.pallas_call(
        mm,
        grid=(n // blk,),
        in_specs=[
            pl.BlockSpec((blk, k), lambda i: (i, 0)),
            pl.BlockSpec((k, feat), lambda i: (0, 0)),
            pl.BlockSpec((1, feat), lambda i: (0, 0)),
        ],
        out_specs=pl.BlockSpec((blk, feat), lambda i: (i, 0)),
        out_shape=jax.ShapeDtypeStruct((n, feat), jnp.float32),
    )


def kernel(x_idx, y_idx, t_idx, A, B, C, W, b):
    rank, n = x_idx.shape
    table_rows = A.shape[1]
    feat = W.shape[0]
    total = rank * n

    xf = x_idx.T.reshape(total)
    yf = y_idx.T.reshape(total)
    tf = t_idx.T.reshape(total)
    At = A.T.astype(jnp.bfloat16)
    Bt = B.T.astype(jnp.bfloat16)
    Ct = C.T.astype(jnp.bfloat16)

    p = _sc_gather_prod(total, rank, table_rows)(xf, yf, tf, At, Bt, Ct)

    w3 = jnp.broadcast_to(W.T[:, None, :], (rank, rank, feat)).reshape(rank * rank, feat)
    w3 = w3.astype(jnp.bfloat16)
    return _tc_linear(n, rank * rank, feat)(
        p.reshape(n, rank * rank), w3, b.reshape(1, feat)
    )
